# Initial kernel scaffold; baseline (speedup 1.0000x reference)
import jax
import jax.numpy as jnp
from jax.experimental import pallas as pl


def kernel(x, c, embed, params):
    raise NotImplementedError("write your pallas kernel here")


# speedup vs baseline: 1.1992x; 1.1992x over previous
import functools
import math

import numpy as np
import jax
import jax.numpy as jnp
from jax import lax
from jax.experimental import pallas as pl
from jax.experimental.pallas import tpu as pltpu
from jax.experimental.pallas import tpu_sc as plsc

S, D, H, F = 2048, 768, 12, 3072
DH = D // H
EPS = 1e-6


@functools.lru_cache(maxsize=1)
def _pe():
    pos = np.arange(S)[:, None].astype(np.float64)
    i = np.arange(D)[None, :]
    ang = pos / np.power(10000.0, (2 * (i // 2)) / D)
    pe = np.where(i % 2 == 0, np.sin(ang), np.cos(ang))
    return jnp.asarray(pe, jnp.float32)


def _sc_gather(embed, idx):
    info = plsc.get_sparse_core_info()
    nc, ns = info.num_cores, info.num_subcores
    nw = nc * ns
    bpw = S // nw
    mesh = plsc.VectorSubcoreMesh(core_axis_name="c", subcore_axis_name="s")

    @functools.partial(
        pl.kernel,
        mesh=mesh,
        out_type=jax.ShapeDtypeStruct((S, D), jnp.float32),
        scratch_types=[
            pltpu.VMEM((bpw,), jnp.int32),
            pltpu.VMEM((bpw, D), jnp.float32),
            pltpu.SemaphoreType.DMA,
        ],
    )
    def k(table_hbm, idx_hbm, out_hbm, idx_v, rows_v, sem):
        wid = lax.axis_index("s") * nc + lax.axis_index("c")
        base = wid * bpw
        pltpu.sync_copy(idx_hbm.at[pl.ds(base, bpw)], idx_v)
        pltpu.async_copy(table_hbm.at[idx_v], rows_v, sem).wait()
        pltpu.sync_copy(rows_v, out_hbm.at[pl.ds(base, bpw)])

    return k(embed, idx)


def _ln(xb, g, b):
    m = jnp.mean(xb, axis=-1, keepdims=True)
    v = jnp.mean((xb - m) * (xb - m), axis=-1, keepdims=True)
    return (xb - m) / jnp.sqrt(v + EPS) * g + b


def _full(shape):
    return pl.BlockSpec(shape, lambda *_: tuple(0 for _ in shape))


def _add_pe(e, pe):
    BS = 512

    def body(e_ref, p_ref, o_ref):
        o_ref[...] = e_ref[...] + p_ref[...]

    return pl.pallas_call(
        body,
        grid=(S // BS,),
        in_specs=[
            pl.BlockSpec((BS, D), lambda i: (i, 0)),
            pl.BlockSpec((BS, D), lambda i: (i, 0)),
        ],
        out_specs=pl.BlockSpec((BS, D), lambda i: (i, 0)),
        out_shape=jax.ShapeDtypeStruct((S, D), jnp.float32),
    )(e, pe)


def _qkv(h, kv, g, b, Wq, bq, Wk, bk, Wv, bv, ln_kv):
    BS = 512

    def body(h_ref, kv_ref, g_ref, b_ref, wq_ref, bq_ref, wk_ref, bk_ref,
             wv_ref, bv_ref, q_ref, k_ref, v_ref):
        hn = _ln(h_ref[...], g_ref[...], b_ref[...])
        kvin = hn if ln_kv else kv_ref[...]
        q_ref[...] = jnp.dot(hn, wq_ref[...],
                             preferred_element_type=jnp.float32) + bq_ref[...]
        k_ref[...] = jnp.dot(kvin, wk_ref[...],
                             preferred_element_type=jnp.float32) + bk_ref[...]
        v_ref[...] = jnp.dot(kvin, wv_ref[...],
                             preferred_element_type=jnp.float32) + bv_ref[...]

    row = pl.BlockSpec((BS, D), lambda i: (i, 0))
    outs = [jax.ShapeDtypeStruct((S, D), jnp.float32)] * 3
    return pl.pallas_call(
        body,
        grid=(S // BS,),
        in_specs=[row, row, _full((1, D)), _full((1, D)),
                  _full((D, D)), _full((1, D)), _full((D, D)), _full((1, D)),
                  _full((D, D)), _full((1, D))],
        out_specs=[row, row, row],
        out_shape=outs,
    )(h, kv, g, b, Wq, bq, Wk, bk, Wv, bv)


def _attention(q, k, v, causal):
    BQ = 256
    HP = 2
    BW = HP * DH
    scale = 1.0 / math.sqrt(DH)

    def body(q_ref, k_ref, v_ref, o_ref):
        qf = q_ref[...]
        kf = k_ref[...]
        vf = v_ref[...]
        if causal:
            i = pl.program_id(1)
            rows = i * BQ + lax.broadcasted_iota(jnp.int32, (BQ, S), 0)
            cols = lax.broadcasted_iota(jnp.int32, (BQ, S), 1)
            keep = cols <= rows
        outs = []
        for hh in range(HP):
            qb = qf[:, hh * DH:(hh + 1) * DH]
            kb = kf[:, hh * DH:(hh + 1) * DH]
            vb = vf[:, hh * DH:(hh + 1) * DH]
            att = lax.dot_general(qb, kb, (((1,), (1,)), ((), ())),
                                  preferred_element_type=jnp.float32) * scale
            if causal:
                att = jnp.where(keep, att, jnp.float32(-1e9))
            att = att - jnp.max(att, axis=-1, keepdims=True)
            att = jnp.exp(att)
            att = att / jnp.sum(att, axis=-1, keepdims=True)
            outs.append(jnp.dot(att, vb, preferred_element_type=jnp.float32))
        o_ref[...] = jnp.concatenate(outs, axis=1)

    return pl.pallas_call(
        body,
        grid=(H // HP, S // BQ),
        in_specs=[
            pl.BlockSpec((BQ, BW), lambda h, i: (i, h)),
            pl.BlockSpec((S, BW), lambda h, i: (0, h)),
            pl.BlockSpec((S, BW), lambda h, i: (0, h)),
        ],
        out_specs=pl.BlockSpec((BQ, BW), lambda h, i: (i, h)),
        out_shape=jax.ShapeDtypeStruct((S, D), jnp.float32),
    )(q, k, v)


def _proj_res(a, W, b, res):
    BS = 512

    def body(a_ref, w_ref, b_ref, r_ref, o_ref):
        o_ref[...] = (jnp.dot(a_ref[...], w_ref[...],
                              preferred_element_type=jnp.float32)
                      + b_ref[...] + r_ref[...])

    row = pl.BlockSpec((BS, D), lambda i: (i, 0))
    return pl.pallas_call(
        body,
        grid=(S // BS,),
        in_specs=[row, _full((D, D)), _full((1, D)), row],
        out_specs=row,
        out_shape=jax.ShapeDtypeStruct((S, D), jnp.float32),
    )(a, W, b, res)


def _ffn1(h, g, b, W1, b1):
    BS, BN = 512, 1536

    def body(h_ref, g_ref, b_ref, w_ref, b1_ref, o_ref):
        hn = _ln(h_ref[...], g_ref[...], b_ref[...])
        o_ref[...] = jax.nn.gelu(
            jnp.dot(hn, w_ref[...], preferred_element_type=jnp.float32)
            + b1_ref[...])

    return pl.pallas_call(
        body,
        grid=(F // BN, S // BS),
        in_specs=[
            pl.BlockSpec((BS, D), lambda j, i: (i, 0)),
            _full((1, D)), _full((1, D)),
            pl.BlockSpec((D, BN), lambda j, i: (0, j)),
            pl.BlockSpec((1, BN), lambda j, i: (0, j)),
        ],
        out_specs=pl.BlockSpec((BS, BN), lambda j, i: (i, j)),
        out_shape=jax.ShapeDtypeStruct((S, F), jnp.float32),
    )(h, g, b, W1, b1)


def _ffn2(t, W2, b2, res):
    BS = 256

    def body(t_ref, w_ref, b_ref, r_ref, o_ref):
        o_ref[...] = (jnp.dot(t_ref[...], w_ref[...],
                              preferred_element_type=jnp.float32)
                      + b_ref[...] + r_ref[...])

    return pl.pallas_call(
        body,
        grid=(S // BS,),
        in_specs=[
            pl.BlockSpec((BS, F), lambda i: (i, 0)),
            _full((F, D)), _full((1, D)),
            pl.BlockSpec((BS, D), lambda i: (i, 0)),
        ],
        out_specs=pl.BlockSpec((BS, D), lambda i: (i, 0)),
        out_shape=jax.ShapeDtypeStruct((S, D), jnp.float32),
    )(t, W2, b2, res)


def _final_ln(h, g, b):
    BS = 512

    def body(h_ref, g_ref, b_ref, o_ref):
        o_ref[...] = _ln(h_ref[...], g_ref[...], b_ref[...])

    row = pl.BlockSpec((BS, D), lambda i: (i, 0))
    return pl.pallas_call(
        body,
        grid=(S // BS,),
        in_specs=[row, _full((1, D)), _full((1, D))],
        out_specs=row,
        out_shape=jax.ShapeDtypeStruct((S, D), jnp.float32),
    )(h, g, b)


def kernel(x, c, embed, params):
    p = params
    L = p['Wq_sa'].shape[0]
    xf = x.reshape(-1).astype(jnp.int32)
    e = _sc_gather(embed, xf)
    h = _add_pe(e, _pe())
    cm = c.reshape(S, D)

    def r2(a):
        return a.reshape(1, -1)

    for l in range(L):
        q, k, v = _qkv(h, h, r2(p['g1'][l]), r2(p['b1'][l]),
                       p['Wq_sa'][l], r2(p['bq_sa'][l]),
                       p['Wk_sa'][l], r2(p['bk_sa'][l]),
                       p['Wv_sa'][l], r2(p['bv_sa'][l]), ln_kv=True)
        a = _attention(q, k, v, causal=True)
        h = _proj_res(a, p['Wo_sa'][l], r2(p['bo_sa'][l]), h)
        q, k, v = _qkv(h, cm, r2(p['g2'][l]), r2(p['b2'][l]),
                       p['Wq_ca'][l], r2(p['bq_ca'][l]),
                       p['Wk_ca'][l], r2(p['bk_ca'][l]),
                       p['Wv_ca'][l], r2(p['bv_ca'][l]), ln_kv=False)
        a = _attention(q, k, v, causal=False)
        h = _proj_res(a, p['Wo_ca'][l], r2(p['bo_ca'][l]), h)
        t = _ffn1(h, r2(p['g3'][l]), r2(p['b3'][l]), p['W1'][l], r2(p['b1f'][l]))
        h = _ffn2(t, p['W2'][l], r2(p['b2f'][l]), h)

    out = _final_ln(h, r2(p['gf']), r2(p['bf']))
    return out.reshape(1, S, D)
